# baseline (speedup 1.0000x reference)
import functools

import jax
import jax.numpy as jnp
from jax import lax
from jax.experimental import pallas as pl
from jax.experimental.pallas import tpu as pltpu
from jax.experimental.pallas import tpu_sc as plsc

D_MODEL = 2048
NUM_CORES = 2
NUM_SUBCORES = 16
NUM_WORKERS = NUM_CORES * NUM_SUBCORES
B_TOTAL = 4 * 4096
SC_ROWS = 8192
TC_ROWS = B_TOTAL - SC_ROWS
SC_PER_W = SC_ROWS // NUM_WORKERS
TC_BLOCK = 1024


_mesh = plsc.VectorSubcoreMesh(core_axis_name="c", subcore_axis_name="s")


@functools.partial(
    pl.kernel,
    mesh=_mesh,
    out_type=jax.ShapeDtypeStruct((SC_ROWS, D_MODEL), jnp.float32),
    scratch_types=[
        pltpu.VMEM((SC_PER_W,), jnp.int32),
        pltpu.VMEM((3, D_MODEL), jnp.float32),
        pltpu.SemaphoreType.DMA,
    ],
)
def _sc_lookup(table_hbm, idx_hbm, out_hbm, idx_v, table_v, sem_s):
    wid = lax.axis_index("s") * NUM_CORES + lax.axis_index("c")
    base = wid * SC_PER_W
    pltpu.sync_copy(idx_hbm.at[pl.ds(base, SC_PER_W)], idx_v)
    pltpu.sync_copy(table_hbm, table_v)

    def group(g, carry):
        v = idx_v[pl.ds(g * 16, 16)]
        handles = []
        for l in range(16):
            r = v[l]
            handles.append(pltpu.async_copy(
                table_v.at[pl.ds(r, 1)],
                out_hbm.at[pl.ds(base + g * 16 + l, 1)], sem_s))
        for h in handles:
            h.wait()
        return carry

    lax.fori_loop(0, SC_PER_W // 16, group, 0)


def _tc_body(idx_ref, table_ref, out_ref):
    idx = idx_ref[...]
    t = table_ref[...]
    out_ref[...] = jnp.where(
        idx == 0, t[0:1, :], jnp.where(idx == 1, t[1:2, :], t[2:3, :]))


_tc_lookup = pl.pallas_call(
    _tc_body,
    grid=(TC_ROWS // TC_BLOCK,),
    in_specs=[
        pl.BlockSpec((TC_BLOCK, 1), lambda i: (i, 0)),
        pl.BlockSpec((3, D_MODEL), lambda i: (0, 0)),
    ],
    out_specs=pl.BlockSpec((TC_BLOCK, D_MODEL), lambda i: (i, 0)),
    out_shape=jax.ShapeDtypeStruct((TC_ROWS, D_MODEL), jnp.float32),
)


def kernel(segment_label, table):
    idx = segment_label.reshape(-1).astype(jnp.int32)
    sc_out = _sc_lookup(table, idx[:SC_ROWS])
    tc_out = _tc_lookup(idx[SC_ROWS:, None], table)
    out = jnp.concatenate([sc_out, tc_out], axis=0)
    return out.reshape(segment_label.shape + (D_MODEL,))


# speedup vs baseline: 2.2208x; 2.2208x over previous
import functools

import jax
import jax.numpy as jnp
from jax import lax
from jax.experimental import pallas as pl
from jax.experimental.pallas import tpu as pltpu
from jax.experimental.pallas import tpu_sc as plsc

D_MODEL = 2048
NUM_CORES = 2
NUM_SUBCORES = 16
NUM_WORKERS = NUM_CORES * NUM_SUBCORES
B_TOTAL = 4 * 4096
B_PER_W = B_TOTAL // NUM_WORKERS
GROUP = 16


_mesh = plsc.VectorSubcoreMesh(core_axis_name="c", subcore_axis_name="s")


@functools.partial(
    pl.kernel,
    mesh=_mesh,
    out_type=jax.ShapeDtypeStruct((B_TOTAL, D_MODEL), jnp.float32),
    scratch_types=[
        pltpu.VMEM((B_PER_W,), jnp.int32),
        pltpu.VMEM((3, D_MODEL), jnp.float32),
        pltpu.SemaphoreType.DMA,
    ],
)
def _sc_lookup(table_hbm, idx_hbm, out_hbm, idx_v, table_v, sem_s):
    wid = lax.axis_index("s") * NUM_CORES + lax.axis_index("c")
    base = wid * B_PER_W
    pltpu.sync_copy(idx_hbm.at[pl.ds(base, B_PER_W)], idx_v)
    pltpu.sync_copy(table_hbm, table_v)

    def drain_one_row():
        pltpu.make_async_copy(
            table_hbm.at[pl.ds(0, 1)], table_v.at[pl.ds(0, 1)], sem_s).wait()

    def group(g, carry):
        v = idx_v[pl.ds(g * GROUP, 16)]
        for l in range(GROUP):
            r = v[l]
            pltpu.async_copy(
                table_v.at[pl.ds(r, 1)],
                out_hbm.at[pl.ds(base + g * GROUP + l, 1)], sem_s)

        @pl.when(g >= 1)
        def _():
            for _l in range(GROUP):
                drain_one_row()

        return carry

    lax.fori_loop(0, B_PER_W // GROUP, group, 0)
    for _l in range(GROUP):
        drain_one_row()


def kernel(segment_label, table):
    idx = segment_label.reshape(-1).astype(jnp.int32)
    out = _sc_lookup(table, idx)
    return out.reshape(segment_label.shape + (D_MODEL,))
